# baseline (speedup 1.0000x reference)
import jax
import jax.numpy as jnp
from jax.experimental import pallas as pl
from jax.experimental.pallas import tpu as pltpu


def _body(pos_ref, hid_ref, emb_ref, out_ref):
    nct = emb_ref.shape[1]
    out_ref[0, :nct, :] = emb_ref[0]
    out_ref[0, nct:, :] = hid_ref[0]


def _mask_body(mask_ref, mask_out_ref):
    nct = mask_out_ref.shape[2] - mask_ref.shape[2]
    mask_out_ref[0, 0, :nct] = jnp.ones((nct,), mask_out_ref.dtype)
    mask_out_ref[0, 0, nct:] = mask_ref[0, 0]


def kernel(hidden_states, attention_mask, embeddings, position):
    B, S, D = hidden_states.shape
    _, NCT, _ = embeddings.shape
    pos = jnp.asarray(position, jnp.int32).reshape((1,))

    grid_spec = pltpu.PrefetchScalarGridSpec(
        num_scalar_prefetch=1,
        grid=(B,),
        in_specs=[
            pl.BlockSpec((1, S, D), lambda b, p: (b, 0, 0)),
            pl.BlockSpec((1, NCT, D), lambda b, p: (p[0], 0, 0)),
        ],
        out_specs=[
            pl.BlockSpec((1, NCT + S, D), lambda b, p: (b, 0, 0)),
        ],
    )

    (out_hid,) = pl.pallas_call(
        _body,
        grid_spec=grid_spec,
        compiler_params=pltpu.CompilerParams(dimension_semantics=("parallel",)),
        out_shape=[
            jax.ShapeDtypeStruct((B, NCT + S, D), hidden_states.dtype),
        ],
    )(pos, hidden_states, embeddings)

    mask3 = attention_mask.reshape(B, 1, S)
    out_mask = pl.pallas_call(
        _mask_body,
        grid=(B,),
        in_specs=[pl.BlockSpec((1, 1, S), lambda b: (b, 0, 0))],
        out_specs=pl.BlockSpec((1, 1, NCT + S), lambda b: (b, 0, 0)),
        out_shape=jax.ShapeDtypeStruct((B, 1, NCT + S), attention_mask.dtype),
    )(mask3)
    return (out_hid, out_mask.reshape(B, NCT + S))


# speedup vs baseline: 1.0390x; 1.0390x over previous
import jax
import jax.numpy as jnp
from jax.experimental import pallas as pl
from jax.experimental.pallas import tpu as pltpu


def _body(pos_ref, hid_ref, mask_ref, emb_ref, out_ref, mask_out_ref):
    nct = emb_ref.shape[1]
    out_ref[0] = jnp.concatenate([emb_ref[0], hid_ref[0]], axis=0)
    mask_out_ref[0, 0, :nct] = jnp.ones((nct,), mask_out_ref.dtype)
    mask_out_ref[0, 0, nct:] = mask_ref[0, 0]


def kernel(hidden_states, attention_mask, embeddings, position):
    B, S, D = hidden_states.shape
    _, NCT, _ = embeddings.shape
    pos = jnp.asarray(position, jnp.int32).reshape((1,))
    mask3 = attention_mask.reshape(B, 1, S)

    grid_spec = pltpu.PrefetchScalarGridSpec(
        num_scalar_prefetch=1,
        grid=(B,),
        in_specs=[
            pl.BlockSpec((1, S, D), lambda b, p: (b, 0, 0)),
            pl.BlockSpec((1, 1, S), lambda b, p: (b, 0, 0)),
            pl.BlockSpec((1, NCT, D), lambda b, p: (p[0], 0, 0)),
        ],
        out_specs=[
            pl.BlockSpec((1, NCT + S, D), lambda b, p: (b, 0, 0)),
            pl.BlockSpec((1, 1, NCT + S), lambda b, p: (b, 0, 0)),
        ],
    )

    out_hid, out_mask = pl.pallas_call(
        _body,
        grid_spec=grid_spec,
        compiler_params=pltpu.CompilerParams(dimension_semantics=("parallel",)),
        out_shape=[
            jax.ShapeDtypeStruct((B, NCT + S, D), hidden_states.dtype),
            jax.ShapeDtypeStruct((B, 1, NCT + S), attention_mask.dtype),
        ],
    )(pos, hidden_states, mask3, embeddings)
    return (out_hid, out_mask.reshape(B, NCT + S))
